# baseline (speedup 1.0000x reference)
import dataclasses
import functools

import jax
import jax.numpy as jnp
from jax import lax
from jax.experimental import pallas as pl
from jax.experimental.pallas import tpu as pltpu
from jax.experimental.pallas import tpu_sc as plsc

_VOCAB = 1000000
_EMBED = 16
_BATCH = 16384
_HIST = 200

_NC, _NS, _L = 2, 16, 16
_NW = _NC * _NS
_WPT = _BATCH // _NW
_CB = _WPT // 128
_JC = 4
_NJ = _HIST // _JC


_RT = 16384
_TGRID = (_VOCAB + _RT - 1) // _RT
_VPAD = _TGRID * _RT


def _tv_body(t_ref, wd_ref, b_ref, o_ref):
    t = t_ref[...]
    s = jnp.sum(t * wd_ref[...], axis=1) + b_ref[0, 0]
    o_ref[...] = s.reshape(_RT // 128, 128)


def _compute_tv(table, W, b):
    tv2 = pl.pallas_call(
        _tv_body,
        grid=(_TGRID,),
        in_specs=[
            pl.BlockSpec((_RT, _EMBED), lambda i: (i, 0)),
            pl.BlockSpec((1, _EMBED), lambda i: (0, 0)),
            pl.BlockSpec(memory_space=pltpu.SMEM),
        ],
        out_specs=pl.BlockSpec((_RT // 128, 128), lambda i: (i, 0)),
        out_shape=jax.ShapeDtypeStruct((_VPAD // 128, 128), jnp.float32),
    )(table, W, b.reshape(1, 1))
    return tv2.reshape(_VPAD)


_XPW = 256
_XRB = 1024


def _padx_body(x_ref, o_ref):
    o_ref[:, 0:_HIST] = x_ref[...]


def _pad_x(x):
    return pl.pallas_call(
        _padx_body,
        grid=(_BATCH // _XRB,),
        in_specs=[pl.BlockSpec((_XRB, _HIST), lambda i: (i, 0))],
        out_specs=pl.BlockSpec((_XRB, _XPW), lambda i: (i, 0)),
        out_shape=jax.ShapeDtypeStruct((_BATCH, _XPW), jnp.int32),
    )(x)


_RC = 16
_NCH = _WPT // _RC
_VW = 208


def _sc_pool(x_hbm, tv_hbm, o_hbm, idx_v, val_v, srow_v, osum_v, sem):
    wid = lax.axis_index("s") * _NC + lax.axis_index("c")
    base = wid * _WPT

    zero = jnp.zeros((_L,), jnp.float32)
    for r in range(_RC):
        val_v[r, pl.ds(_VW - _L, _L)] = zero

    iota16 = lax.iota(jnp.int32, _L) * _L

    @pl.loop(0, _NCH)
    def _(ch):
        pltpu.sync_copy(x_hbm.at[pl.ds(base + ch * _RC, _RC), :], idx_v)
        for r in range(_RC):
            pltpu.make_async_copy(
                tv_hbm.at[idx_v.at[r, pl.ds(0, 128)]],
                val_v.at[r, pl.ds(0, 128)], sem,
            ).start()
            pltpu.make_async_copy(
                tv_hbm.at[idx_v.at[r, pl.ds(128, _HIST - 128)]],
                val_v.at[r, pl.ds(128, _HIST - 128)], sem,
            ).start()
        for r in range(_RC):
            pltpu.make_async_copy(
                tv_hbm.at[idx_v.at[r, pl.ds(0, 128)]],
                val_v.at[r, pl.ds(0, 128)], sem,
            ).wait()
            pltpu.make_async_copy(
                tv_hbm.at[idx_v.at[r, pl.ds(128, _HIST - 128)]],
                val_v.at[r, pl.ds(128, _HIST - 128)], sem,
            ).wait()
        for r in range(_RC):
            s = val_v[r, pl.ds(0, _L)]
            for k in range(1, _VW // _L):
                s = s + val_v[r, pl.ds(k * _L, _L)]
            srow_v[pl.ds(r * _L, _L)] = s
        y = plsc.load_gather(srow_v, [iota16])
        for k in range(1, _L):
            y = y + plsc.load_gather(srow_v, [iota16 + k])
        osum_v[pl.ds(ch * _RC, _RC)] = y

    inv = jnp.float32(1.0 / _HIST)
    one = jnp.float32(1.0)
    for k in range(_WPT // _L):
        sl = pl.ds(k * _L, _L)
        z = osum_v[sl] * inv
        osum_v[sl] = one / (one + jnp.exp(-z))
    pltpu.sync_copy(osum_v, o_hbm.at[wid])


def _sc_compiler_params():
    cp = pltpu.CompilerParams()
    if "needs_layout_passes" in pltpu.CompilerParams.__dataclass_fields__:
        cp = dataclasses.replace(cp, needs_layout_passes=False)
    return cp


def _pooled_probs(x, tv):
    sc = pl.kernel(
        _sc_pool,
        out_type=jax.ShapeDtypeStruct((_NW, _WPT), jnp.float32),
        mesh=plsc.VectorSubcoreMesh(core_axis_name="c", subcore_axis_name="s"),
        compiler_params=_sc_compiler_params(),
        scratch_types=[
            pltpu.VMEM((_RC, _XPW), jnp.int32),
            pltpu.VMEM((_RC, _VW), jnp.float32),
            pltpu.VMEM((_RC * _L,), jnp.float32),
            pltpu.VMEM((_WPT,), jnp.float32),
            pltpu.SemaphoreType.DMA,
        ],
    )
    return sc(x, tv)


def kernel(x, table, W, b):
    tv = _compute_tv(table, W, b)
    probs = _pooled_probs(_pad_x(x), tv)
    return probs.reshape(_BATCH, 1)


# speedup vs baseline: 2.7097x; 2.7097x over previous
import dataclasses
import functools

import jax
import jax.numpy as jnp
from jax import lax
from jax.experimental import pallas as pl
from jax.experimental.pallas import tpu as pltpu
from jax.experimental.pallas import tpu_sc as plsc

_VOCAB = 1000000
_EMBED = 16
_BATCH = 16384
_HIST = 200

_NC, _NS, _L = 2, 16, 16
_NW = _NC * _NS
_WPT = _BATCH // _NW
_JC = 4
_NJ = _HIST // _JC


_CV = 65536
_TG = 16
_VPAD = _TG * _CV


def _tv_body(tT_ref, w_ref, b_ref, o_ref):
    m = tT_ref[...] * w_ref[...]
    s = jnp.sum(m, axis=0) + b_ref[0, 0]
    o_ref[...] = s.reshape(_CV // 128, 128)


def _compute_tv(tableT, wT, b):
    tv2 = pl.pallas_call(
        _tv_body,
        grid=(_TG,),
        in_specs=[
            pl.BlockSpec((_EMBED, _CV), lambda i: (0, i)),
            pl.BlockSpec((_EMBED, 1), lambda i: (0, 0)),
            pl.BlockSpec(memory_space=pltpu.SMEM),
        ],
        out_specs=pl.BlockSpec((_CV // 128, 128), lambda i: (i, 0)),
        out_shape=jax.ShapeDtypeStruct((_VPAD // 128, 128), jnp.float32),
    )(tableT, wT, b.reshape(1, 1))
    return tv2.reshape(_VPAD)


def _sc_pool(xT_hbm, tv_hbm, o_hbm, idx_v, val_v, acc_v, sem):
    wid = lax.axis_index("s") * _NC + lax.axis_index("c")
    base = wid * _WPT

    zero = jnp.zeros((_L,), jnp.float32)
    for k in range(_WPT // _L):
        acc_v[pl.ds(k * _L, _L)] = zero

    @pl.loop(0, _NJ)
    def _(ch):
        pltpu.sync_copy(
            xT_hbm.at[pl.ds(ch * _JC, _JC), pl.ds(base, _WPT)], idx_v
        )
        for j in range(_JC):
            for c in range(_WPT // 128):
                pltpu.make_async_copy(
                    tv_hbm.at[idx_v.at[j, pl.ds(c * 128, 128)]],
                    val_v.at[j, pl.ds(c * 128, 128)], sem,
                ).start()
        for j in range(_JC):
            for c in range(_WPT // 128):
                pltpu.make_async_copy(
                    tv_hbm.at[idx_v.at[j, pl.ds(c * 128, 128)]],
                    val_v.at[j, pl.ds(c * 128, 128)], sem,
                ).wait()
        for k in range(_WPT // _L):
            sl = pl.ds(k * _L, _L)
            s = acc_v[sl]
            for j in range(_JC):
                s = s + val_v[j, sl]
            acc_v[sl] = s

    inv = jnp.float32(1.0 / _HIST)
    one = jnp.float32(1.0)
    for k in range(_WPT // _L):
        sl = pl.ds(k * _L, _L)
        z = acc_v[sl] * inv
        acc_v[sl] = one / (one + jnp.exp(-z))
    pltpu.sync_copy(acc_v, o_hbm.at[wid])


def _sc_compiler_params():
    cp = pltpu.CompilerParams()
    if "needs_layout_passes" in pltpu.CompilerParams.__dataclass_fields__:
        cp = dataclasses.replace(cp, needs_layout_passes=False)
    return cp


def _pooled_probs(xT, tv):
    sc = pl.kernel(
        _sc_pool,
        out_type=jax.ShapeDtypeStruct((_NW, _WPT), jnp.float32),
        mesh=plsc.VectorSubcoreMesh(core_axis_name="c", subcore_axis_name="s"),
        compiler_params=_sc_compiler_params(),
        scratch_types=[
            pltpu.VMEM((_JC, _WPT), jnp.int32),
            pltpu.VMEM((_JC, _WPT), jnp.float32),
            pltpu.VMEM((_WPT,), jnp.float32),
            pltpu.SemaphoreType.DMA,
        ],
    )
    return sc(xT, tv)


def kernel(x, table, W, b):
    tv = _compute_tv(table.T, W.reshape(_EMBED, 1), b)
    probs = _pooled_probs(x.T, tv)
    return probs.reshape(_BATCH, 1)


# speedup vs baseline: 5.0721x; 1.8718x over previous
import dataclasses
import functools

import jax
import jax.numpy as jnp
from jax import lax
from jax.experimental import pallas as pl
from jax.experimental.pallas import tpu as pltpu
from jax.experimental.pallas import tpu_sc as plsc

_VOCAB = 1000000
_EMBED = 16
_BATCH = 16384
_HIST = 200

_NC, _NS, _L = 2, 16, 16
_NW = _NC * _NS
_WPT = _BATCH // _NW
_JC = 4
_NJ = _HIST // _JC


_CV = 65536
_TG = 16
_VPAD = _TG * _CV


def _tv_body(tT_ref, w_ref, b_ref, o_ref):
    m = tT_ref[...] * w_ref[...]
    s = jnp.sum(m, axis=0) + b_ref[0, 0]
    o_ref[...] = s.reshape(_CV // 128, 128)


def _compute_tv(tableT, wT, b):
    tv2 = pl.pallas_call(
        _tv_body,
        grid=(_TG,),
        in_specs=[
            pl.BlockSpec((_EMBED, _CV), lambda i: (0, i)),
            pl.BlockSpec((_EMBED, 1), lambda i: (0, 0)),
            pl.BlockSpec(memory_space=pltpu.SMEM),
        ],
        out_specs=pl.BlockSpec((_CV // 128, 128), lambda i: (i, 0)),
        out_shape=jax.ShapeDtypeStruct((_VPAD // 128, 128), jnp.float32),
    )(tableT, wT, b.reshape(1, 1))
    return tv2.reshape(_VPAD)


def _sc_pool(xT_hbm, tv_hbm, o_hbm, idx_v, val_v, acc_v, tvs_v, sem):
    wid = lax.axis_index("s") * _NC + lax.axis_index("c")
    base = wid * _WPT

    seg = _VPAD // _NS
    sid = lax.axis_index("s")
    pltpu.sync_copy(
        tv_hbm.at[pl.ds(sid * seg, seg)], tvs_v.at[pl.ds(sid * seg, seg)]
    )
    plsc.subcore_barrier()

    zero = jnp.zeros((_L,), jnp.float32)
    for k in range(_WPT // _L):
        acc_v[pl.ds(k * _L, _L)] = zero

    @pl.loop(0, _NJ)
    def _(ch):
        pltpu.sync_copy(
            xT_hbm.at[pl.ds(ch * _JC, _JC), pl.ds(base, _WPT)], idx_v
        )
        for j in range(_JC):
            for c in range(_WPT // 128):
                pltpu.make_async_copy(
                    tvs_v.at[idx_v.at[j, pl.ds(c * 128, 128)]],
                    val_v.at[j, pl.ds(c * 128, 128)], sem,
                ).start()
        for j in range(_JC):
            for c in range(_WPT // 128):
                pltpu.make_async_copy(
                    tvs_v.at[idx_v.at[j, pl.ds(c * 128, 128)]],
                    val_v.at[j, pl.ds(c * 128, 128)], sem,
                ).wait()
        for k in range(_WPT // _L):
            sl = pl.ds(k * _L, _L)
            s = acc_v[sl]
            for j in range(_JC):
                s = s + val_v[j, sl]
            acc_v[sl] = s

    inv = jnp.float32(1.0 / _HIST)
    one = jnp.float32(1.0)
    for k in range(_WPT // _L):
        sl = pl.ds(k * _L, _L)
        z = acc_v[sl] * inv
        acc_v[sl] = one / (one + jnp.exp(-z))
    pltpu.sync_copy(acc_v, o_hbm.at[wid])


def _sc_compiler_params():
    cp = pltpu.CompilerParams()
    if "needs_layout_passes" in pltpu.CompilerParams.__dataclass_fields__:
        cp = dataclasses.replace(cp, needs_layout_passes=False)
    return cp


def _pooled_probs(xT, tv):
    sc = pl.kernel(
        _sc_pool,
        out_type=jax.ShapeDtypeStruct((_NW, _WPT), jnp.float32),
        mesh=plsc.VectorSubcoreMesh(core_axis_name="c", subcore_axis_name="s"),
        compiler_params=_sc_compiler_params(),
        scratch_types=[
            pltpu.VMEM((_JC, _WPT), jnp.int32),
            pltpu.VMEM((_JC, _WPT), jnp.float32),
            pltpu.VMEM((_WPT,), jnp.float32),
            pltpu.VMEM_SHARED((_VPAD,), jnp.float32),
            pltpu.SemaphoreType.DMA,
        ],
    )
    return sc(xT, tv)


def kernel(x, table, W, b):
    tv = _compute_tv(table.T, W.reshape(_EMBED, 1), b)
    probs = _pooled_probs(x.T, tv)
    return probs.reshape(_BATCH, 1)


# speedup vs baseline: 5.3630x; 1.0574x over previous
import dataclasses
import functools

import jax
import jax.numpy as jnp
from jax import lax
from jax.experimental import pallas as pl
from jax.experimental.pallas import tpu as pltpu
from jax.experimental.pallas import tpu_sc as plsc

_VOCAB = 1000000
_EMBED = 16
_BATCH = 16384
_HIST = 200

_NC, _NS, _L = 2, 16, 16
_NW = _NC * _NS
_WPT = _BATCH // _NW
_JC = 2
_NJ = _HIST // _JC


_CV = 32768
_TG = (_VOCAB + _CV - 1) // _CV
_VPAD = _TG * _CV


def _tv_body(tT_ref, w_ref, b_ref, o_ref):
    m = tT_ref[...] * w_ref[...]
    s = jnp.sum(m, axis=0) + b_ref[0, 0]
    o_ref[...] = s.reshape(_CV // 128, 128)


def _compute_tv(tableT, wT, b):
    tv2 = pl.pallas_call(
        _tv_body,
        grid=(_TG,),
        in_specs=[
            pl.BlockSpec((_EMBED, _CV), lambda i: (0, i)),
            pl.BlockSpec((_EMBED, 1), lambda i: (0, 0)),
            pl.BlockSpec(memory_space=pltpu.SMEM),
        ],
        out_specs=pl.BlockSpec((_CV // 128, 128), lambda i: (i, 0)),
        out_shape=jax.ShapeDtypeStruct((_VPAD // 128, 128), jnp.float32),
    )(tableT, wT, b.reshape(1, 1))
    return tv2.reshape(_VPAD)


def _sc_pool(xT_hbm, tv_hbm, o_hbm, idx_v, val_v, acc_v, tvs_v, semg, semi):
    wid = lax.axis_index("s") * _NC + lax.axis_index("c")
    base = wid * _WPT
    seg = _VPAD // _NS
    sid = lax.axis_index("s")

    def idx_copy(ch, p):
        return pltpu.make_async_copy(
            xT_hbm.at[pl.ds(ch * _JC, _JC), pl.ds(base, _WPT)],
            idx_v.at[p], semi,
        )

    def gathers(p):
        for j in range(_JC):
            for c in range(_WPT // 128):
                yield pltpu.make_async_copy(
                    tvs_v.at[idx_v.at[p, j, pl.ds(c * 128, 128)]],
                    val_v.at[p, j, pl.ds(c * 128, 128)], semg,
                )

    idx_copy(0, 0).start()
    pltpu.sync_copy(
        tv_hbm.at[pl.ds(sid * seg, seg)], tvs_v.at[pl.ds(sid * seg, seg)]
    )
    plsc.subcore_barrier()

    zero = jnp.zeros((_L,), jnp.float32)
    for k in range(_WPT // _L):
        acc_v[pl.ds(k * _L, _L)] = zero

    idx_copy(0, 0).wait()
    for cp in gathers(0):
        cp.start()
    idx_copy(1, 1).start()

    def accum(p):
        for k in range(_WPT // _L):
            sl = pl.ds(k * _L, _L)
            s = acc_v[sl]
            for j in range(_JC):
                s = s + val_v[p, j, sl]
            acc_v[sl] = s

    def phase(ch, p, q, prefetch):
        for cp in gathers(p):
            cp.wait()
        idx_copy(ch + 1, q).wait()
        for cp in gathers(q):
            cp.start()
        if prefetch:
            idx_copy(ch + 2, p).start()
        accum(p)

    @pl.loop(0, _NJ // 2 - 1)
    def _(i):
        ch = i * 2
        phase(ch, 0, 1, prefetch=True)
        phase(ch + 1, 1, 0, prefetch=True)

    phase(_NJ - 2, 0, 1, prefetch=False)
    for cp in gathers(1):
        cp.wait()
    accum(1)

    inv = jnp.float32(1.0 / _HIST)
    one = jnp.float32(1.0)
    for k in range(_WPT // _L):
        sl = pl.ds(k * _L, _L)
        z = acc_v[sl] * inv
        acc_v[sl] = one / (one + jnp.exp(-z))
    pltpu.sync_copy(acc_v, o_hbm.at[wid])


def _sc_compiler_params():
    cp = pltpu.CompilerParams()
    if "needs_layout_passes" in pltpu.CompilerParams.__dataclass_fields__:
        cp = dataclasses.replace(cp, needs_layout_passes=False)
    return cp


def _pooled_probs(xT, tv):
    sc = pl.kernel(
        _sc_pool,
        out_type=jax.ShapeDtypeStruct((_NW, _WPT), jnp.float32),
        mesh=plsc.VectorSubcoreMesh(core_axis_name="c", subcore_axis_name="s"),
        compiler_params=_sc_compiler_params(),
        scratch_types=[
            pltpu.VMEM((2, _JC, _WPT), jnp.int32),
            pltpu.VMEM((2, _JC, _WPT), jnp.float32),
            pltpu.VMEM((_WPT,), jnp.float32),
            pltpu.VMEM_SHARED((_VPAD,), jnp.float32),
            pltpu.SemaphoreType.DMA,
            pltpu.SemaphoreType.DMA,
        ],
    )
    return sc(xT, tv)


def kernel(x, table, W, b):
    tv = _compute_tv(table.T, W.reshape(_EMBED, 1), b)
    probs = _pooled_probs(x.T, tv)
    return probs.reshape(_BATCH, 1)
